# baseline (speedup 1.0000x reference)
import functools

import jax
import jax.numpy as jnp
from jax import lax
from jax.experimental import pallas as pl
from jax.experimental.pallas import tpu as pltpu
from jax.experimental.pallas import tpu_sc as plsc

B = 4096
L = 200
D = 512
NCLS = 1000
VPAD = 1024

NW = 32
NHALF = 4
BH = B // NHALF
CH = BH // NW
KSH = 128 // CH


def _sc_histogram_half(xt, half):
    mesh = plsc.VectorSubcoreMesh(core_axis_name="c", subcore_axis_name="s")

    @functools.partial(
        pl.kernel,
        mesh=mesh,
        out_type=jax.ShapeDtypeStruct((BH, VPAD), jnp.float32),
        scratch_types=[
            pltpu.VMEM((L, 128), jnp.int32),
            pltpu.VMEM((CH, VPAD), jnp.float32),
        ],
        compiler_params=pltpu.CompilerParams(needs_layout_passes=False),
    )
    def hist_kernel(xt_hbm, counts_hbm, idx_v, hist_v):
        wid = lax.axis_index("c") * 16 + lax.axis_index("s")
        ones = jnp.ones((16,), jnp.float32)
        zeros = jnp.zeros((16,), jnp.float32)
        lane = lax.iota(jnp.int32, 16)

        aligned = half * BH + (wid // KSH) * 128
        off = (wid % KSH) * CH
        pltpu.sync_copy(xt_hbm.at[:, pl.ds(aligned, 128)], idx_v)

        def zero_row(r, carry):
            for k in range(VPAD // 16):
                hist_v[r, pl.ds(k * 16, 16)] = zeros
            return carry

        lax.fori_loop(0, CH, zero_row, 0)

        def do_tok(j, carry):
            for g in range(CH // 16):
                rvec = lane + (g * 16)
                ids = idx_v[j, pl.ds(off + g * 16, 16)]
                plsc.addupdate_scatter(hist_v, [rvec, ids], ones)
            return carry

        lax.fori_loop(0, L, do_tok, 0)

        pltpu.sync_copy(hist_v, counts_hbm.at[pl.ds(wid * CH, CH)])

    return hist_kernel(xt)


BB = 256
GH = BH // BB


def _mlp_body(_, counts_ref, emb_ref, w1_ref, b1_ref, w2t_ref, b2_ref, outt_ref):
    c = counts_ref[...]
    denom = 200.0 - c[:, 0:1]
    cv = c[:, :NCLS]
    col = lax.broadcasted_iota(jnp.int32, (BB, NCLS), 1)
    cv = jnp.where(col == 0, 0.0, cv)
    s = jnp.dot(
        cv.astype(jnp.bfloat16),
        emb_ref[...].astype(jnp.bfloat16),
        preferred_element_type=jnp.float32,
    )
    pooled = s / denom
    h = jnp.dot(
        pooled.astype(jnp.bfloat16),
        w1_ref[...].astype(jnp.bfloat16),
        preferred_element_type=jnp.float32,
    ) + b1_ref[...]
    h = jnp.maximum(h, 0.0)
    outt_ref[...] = lax.dot_general(
        w2t_ref[...].astype(jnp.bfloat16),
        h.astype(jnp.bfloat16),
        (((1,), (1,)), ((), ())),
        preferred_element_type=jnp.float32,
    ) + b2_ref[...]


def _tc_mlp_half(prev_outt, counts_h, emb, W1, b1r, W2t, b2r, half):
    body = _mlp_body if prev_outt is not None else (
        lambda c, e, w1, bb1, w2t, bb2, o: _mlp_body(None, c, e, w1, bb1, w2t, bb2, o)
    )
    specs = [
        pl.BlockSpec((BB, VPAD), lambda i: (i, 0)),
        pl.BlockSpec((NCLS, D), lambda i: (0, 0)),
        pl.BlockSpec((D, D), lambda i: (0, 0)),
        pl.BlockSpec((1, D), lambda i: (0, 0)),
        pl.BlockSpec((NCLS, D), lambda i: (0, 0)),
        pl.BlockSpec((NCLS, 1), lambda i: (0, 0)),
    ]
    args = (counts_h, emb, W1, b1r, W2t, b2r)
    aliases = {}
    if prev_outt is not None:
        specs = [pl.BlockSpec(memory_space=pl.MemorySpace.ANY)] + specs
        args = (prev_outt,) + args
        aliases = {0: 0}
    return pl.pallas_call(
        body,
        grid=(GH,),
        in_specs=specs,
        out_specs=pl.BlockSpec((NCLS, BB), lambda i, h=half: (0, h * GH + i)),
        out_shape=jax.ShapeDtypeStruct((NCLS, B), jnp.float32),
        input_output_aliases=aliases,
    )(*args)


def kernel(x, emb, W1, b1, W2, b2):
    xt = jnp.swapaxes(x.astype(jnp.int32), 0, 1)
    W2t = jnp.swapaxes(W2, 0, 1)
    b1r = b1.reshape(1, D)
    b2r = b2.reshape(NCLS, 1)

    counts = [_sc_histogram_half(xt, h) for h in range(NHALF)]
    outt = _tc_mlp_half(None, counts[0], emb, W1, b1r, W2t, b2r, 0)
    for h in range(1, NHALF):
        outt = _tc_mlp_half(outt, counts[h], emb, W1, b1r, W2t, b2r, h)
    return outt.T


# speedup vs baseline: 1.1805x; 1.1805x over previous
import functools

import jax
import jax.numpy as jnp
from jax import lax
from jax.experimental import pallas as pl
from jax.experimental.pallas import tpu as pltpu
from jax.experimental.pallas import tpu_sc as plsc

B = 4096
L = 200
D = 512
NCLS = 1000
VPAD = 1024

NW = 32
NHALF = 2
BH = B // NHALF
CH = BH // NW
KSH = 128 // CH


def _sc_histogram_half(xt, half):
    mesh = plsc.VectorSubcoreMesh(core_axis_name="c", subcore_axis_name="s")

    @functools.partial(
        pl.kernel,
        mesh=mesh,
        out_type=jax.ShapeDtypeStruct((BH, VPAD), jnp.float32),
        scratch_types=[
            pltpu.VMEM((L, 128), jnp.int32),
            pltpu.VMEM((CH, VPAD), jnp.float32),
        ],
        compiler_params=pltpu.CompilerParams(needs_layout_passes=False),
    )
    def hist_kernel(xt_hbm, counts_hbm, idx_v, hist_v):
        wid = lax.axis_index("c") * 16 + lax.axis_index("s")
        ones = jnp.ones((16,), jnp.float32)
        zeros = jnp.zeros((16,), jnp.float32)
        lane = lax.iota(jnp.int32, 16)

        aligned = half * BH + (wid // KSH) * 128
        off = (wid % KSH) * CH
        pltpu.sync_copy(xt_hbm.at[:, pl.ds(aligned, 128)], idx_v)

        def zero_row(r, carry):
            for k in range(VPAD // 16):
                hist_v[r, pl.ds(k * 16, 16)] = zeros
            return carry

        lax.fori_loop(0, CH, zero_row, 0)

        def do_tok(j, carry):
            for g in range(CH // 16):
                rvec = lane + (g * 16)
                ids = idx_v[j, pl.ds(off + g * 16, 16)]
                plsc.addupdate_scatter(hist_v, [rvec, ids], ones)
            return carry

        lax.fori_loop(0, L, do_tok, 0)

        pltpu.sync_copy(hist_v, counts_hbm.at[pl.ds(wid * CH, CH)])

    return hist_kernel(xt)


BB = 256
GH = BH // BB


def _mlp_body(_, counts_ref, emb_ref, w1_ref, b1_ref, w2t_ref, b2_ref, outt_ref):
    c = counts_ref[...]
    denom = 200.0 - c[:, 0:1]
    cv = c[:, :NCLS]
    col = lax.broadcasted_iota(jnp.int32, (BB, NCLS), 1)
    cv = jnp.where(col == 0, 0.0, cv)
    s = jnp.dot(
        cv.astype(jnp.bfloat16),
        emb_ref[...].astype(jnp.bfloat16),
        preferred_element_type=jnp.float32,
    )
    pooled = s / denom
    h = jnp.dot(
        pooled.astype(jnp.bfloat16),
        w1_ref[...].astype(jnp.bfloat16),
        preferred_element_type=jnp.float32,
    ) + b1_ref[...]
    h = jnp.maximum(h, 0.0)
    outt_ref[...] = lax.dot_general(
        w2t_ref[...].astype(jnp.bfloat16),
        h.astype(jnp.bfloat16),
        (((1,), (1,)), ((), ())),
        preferred_element_type=jnp.float32,
    ) + b2_ref[...]


def _tc_mlp_half(prev_outt, counts_h, emb, W1, b1r, W2t, b2r, half):
    body = _mlp_body if prev_outt is not None else (
        lambda c, e, w1, bb1, w2t, bb2, o: _mlp_body(None, c, e, w1, bb1, w2t, bb2, o)
    )
    specs = [
        pl.BlockSpec((BB, VPAD), lambda i: (i, 0)),
        pl.BlockSpec((NCLS, D), lambda i: (0, 0)),
        pl.BlockSpec((D, D), lambda i: (0, 0)),
        pl.BlockSpec((1, D), lambda i: (0, 0)),
        pl.BlockSpec((NCLS, D), lambda i: (0, 0)),
        pl.BlockSpec((NCLS, 1), lambda i: (0, 0)),
    ]
    args = (counts_h, emb, W1, b1r, W2t, b2r)
    aliases = {}
    if prev_outt is not None:
        specs = [pl.BlockSpec(memory_space=pl.MemorySpace.ANY)] + specs
        args = (prev_outt,) + args
        aliases = {0: 0}
    return pl.pallas_call(
        body,
        grid=(GH,),
        in_specs=specs,
        out_specs=pl.BlockSpec((NCLS, BB), lambda i, h=half: (0, h * GH + i)),
        out_shape=jax.ShapeDtypeStruct((NCLS, B), jnp.float32),
        input_output_aliases=aliases,
    )(*args)


def kernel(x, emb, W1, b1, W2, b2):
    xt = jnp.swapaxes(x.astype(jnp.int32), 0, 1)
    W2t = jnp.swapaxes(W2, 0, 1)
    b1r = b1.reshape(1, D)
    b2r = b2.reshape(NCLS, 1)

    counts = [_sc_histogram_half(xt, h) for h in range(NHALF)]
    outt = _tc_mlp_half(None, counts[0], emb, W1, b1r, W2t, b2r, 0)
    for h in range(1, NHALF):
        outt = _tc_mlp_half(outt, counts[h], emb, W1, b1r, W2t, b2r, h)
    return outt.T


# speedup vs baseline: 1.3118x; 1.1112x over previous
import functools

import jax
import jax.numpy as jnp
from jax import lax
from jax.experimental import pallas as pl
from jax.experimental.pallas import tpu as pltpu
from jax.experimental.pallas import tpu_sc as plsc

B = 4096
L = 200
D = 512
NCLS = 1000
VPAD = 1024

NW = 32
NHALF = 2
BH = B // NHALF
CH = BH // NW
KSH = 128 // CH


def _sc_histogram_half(xt, half):
    mesh = plsc.VectorSubcoreMesh(core_axis_name="c", subcore_axis_name="s")

    @functools.partial(
        pl.kernel,
        mesh=mesh,
        out_type=jax.ShapeDtypeStruct((BH, VPAD), jnp.float32),
        scratch_types=[
            pltpu.VMEM((L, 128), jnp.int32),
            pltpu.VMEM((CH, VPAD), jnp.float32),
            pltpu.SemaphoreType.DMA,
        ],
        compiler_params=pltpu.CompilerParams(needs_layout_passes=False),
    )
    def hist_kernel(xt_hbm, counts_hbm, idx_v, hist_v, sem):
        wid = lax.axis_index("c") * 16 + lax.axis_index("s")
        ones = jnp.ones((16,), jnp.float32)
        zeros = jnp.zeros((16,), jnp.float32)
        lane = lax.iota(jnp.int32, 16)

        aligned = half * BH + (wid // KSH) * 128
        off = (wid % KSH) * CH
        cp = pltpu.async_copy(xt_hbm.at[:, pl.ds(aligned, 128)], idx_v, sem)

        def zero_row(r, carry):
            for k in range(VPAD // 16):
                hist_v[r, pl.ds(k * 16, 16)] = zeros
            return carry

        lax.fori_loop(0, CH, zero_row, 0)
        cp.wait()

        def do_tok(j, carry):
            for g in range(CH // 16):
                rvec = lane + (g * 16)
                ids = idx_v[j, pl.ds(off + g * 16, 16)]
                plsc.addupdate_scatter(hist_v, [rvec, ids], ones)
            return carry

        lax.fori_loop(0, L, do_tok, 0)

        pltpu.sync_copy(hist_v, counts_hbm.at[pl.ds(wid * CH, CH)])

    return hist_kernel(xt)


BB = 512
GH = BH // BB


def _mlp_body(_, counts_ref, emb_ref, w1_ref, b1_ref, w2t_ref, b2_ref, outt_ref):
    c = counts_ref[...]
    denom = 200.0 - c[:, 0:1]
    cv = c[:, :NCLS]
    col = lax.broadcasted_iota(jnp.int32, (BB, NCLS), 1)
    cv = jnp.where(col == 0, 0.0, cv)
    s = jnp.dot(
        cv.astype(jnp.bfloat16),
        emb_ref[...].astype(jnp.bfloat16),
        preferred_element_type=jnp.float32,
    )
    pooled = s / denom
    h = jnp.dot(
        pooled.astype(jnp.bfloat16),
        w1_ref[...].astype(jnp.bfloat16),
        preferred_element_type=jnp.float32,
    ) + b1_ref[...]
    h = jnp.maximum(h, 0.0)
    outt_ref[...] = lax.dot_general(
        w2t_ref[...].astype(jnp.bfloat16),
        h.astype(jnp.bfloat16),
        (((1,), (1,)), ((), ())),
        preferred_element_type=jnp.float32,
    ) + b2_ref[...]


def _tc_mlp_half(prev_outt, counts_h, emb, W1, b1r, W2t, b2r, half):
    body = _mlp_body if prev_outt is not None else (
        lambda c, e, w1, bb1, w2t, bb2, o: _mlp_body(None, c, e, w1, bb1, w2t, bb2, o)
    )
    specs = [
        pl.BlockSpec((BB, VPAD), lambda i: (i, 0)),
        pl.BlockSpec((NCLS, D), lambda i: (0, 0)),
        pl.BlockSpec((D, D), lambda i: (0, 0)),
        pl.BlockSpec((1, D), lambda i: (0, 0)),
        pl.BlockSpec((NCLS, D), lambda i: (0, 0)),
        pl.BlockSpec((NCLS, 1), lambda i: (0, 0)),
    ]
    args = (counts_h, emb, W1, b1r, W2t, b2r)
    aliases = {}
    if prev_outt is not None:
        specs = [pl.BlockSpec(memory_space=pl.MemorySpace.ANY)] + specs
        args = (prev_outt,) + args
        aliases = {0: 0}
    return pl.pallas_call(
        body,
        grid=(GH,),
        in_specs=specs,
        out_specs=pl.BlockSpec((NCLS, BB), lambda i, h=half: (0, h * GH + i)),
        out_shape=jax.ShapeDtypeStruct((NCLS, B), jnp.float32),
        input_output_aliases=aliases,
    )(*args)


def kernel(x, emb, W1, b1, W2, b2):
    xt = jnp.swapaxes(x.astype(jnp.int32), 0, 1)
    W2t = jnp.swapaxes(W2, 0, 1)
    b1r = b1.reshape(1, D)
    b2r = b2.reshape(NCLS, 1)

    counts = [_sc_histogram_half(xt, h) for h in range(NHALF)]
    outt = _tc_mlp_half(None, counts[0], emb, W1, b1r, W2t, b2r, 0)
    for h in range(1, NHALF):
        outt = _tc_mlp_half(outt, counts[h], emb, W1, b1r, W2t, b2r, h)
    return outt.T


# speedup vs baseline: 1.3517x; 1.0304x over previous
import functools

import jax
import jax.numpy as jnp
from jax import lax
from jax.experimental import pallas as pl
from jax.experimental.pallas import tpu as pltpu
from jax.experimental.pallas import tpu_sc as plsc

B = 4096
L = 200
D = 512
NCLS = 1000
VPAD = 1024

NW = 32
NHALF = 2
BH = B // NHALF
CH = BH // NW
KSH = 128 // CH


def _sc_histogram_half(xt, half):
    mesh = plsc.VectorSubcoreMesh(core_axis_name="c", subcore_axis_name="s")

    @functools.partial(
        pl.kernel,
        mesh=mesh,
        out_type=jax.ShapeDtypeStruct((BH, VPAD), jnp.float32),
        scratch_types=[
            pltpu.VMEM((L, 128), jnp.int32),
            pltpu.VMEM((CH, VPAD), jnp.float32),
            pltpu.SemaphoreType.DMA,
        ],
        compiler_params=pltpu.CompilerParams(needs_layout_passes=False),
    )
    def hist_kernel(xt_hbm, counts_hbm, idx_v, hist_v, sem):
        wid = lax.axis_index("c") * 16 + lax.axis_index("s")
        ones = jnp.ones((16,), jnp.float32)
        zeros = jnp.zeros((16,), jnp.float32)
        lane = lax.iota(jnp.int32, 16)

        aligned = half * BH + (wid // KSH) * 128
        off = (wid % KSH) * CH
        cp = pltpu.async_copy(xt_hbm.at[:, pl.ds(aligned, 128)], idx_v, sem)

        def zero_row(r, carry):
            for k in range(VPAD // 16):
                hist_v[r, pl.ds(k * 16, 16)] = zeros
            return carry

        lax.fori_loop(0, CH, zero_row, 0)
        cp.wait()

        wbs = []
        for g in range(CH // 16):
            rvec = lane + (g * 16)

            def do_tok(j, carry, rvec=rvec, g=g):
                ids = idx_v[j, pl.ds(off + g * 16, 16)]
                plsc.addupdate_scatter(hist_v, [rvec, ids], ones)
                return carry

            lax.fori_loop(0, L, do_tok, 0)
            wbs.append(pltpu.async_copy(
                hist_v.at[pl.ds(g * 16, 16)],
                counts_hbm.at[pl.ds(wid * CH + g * 16, 16)],
                sem,
            ))
        for wb in wbs:
            wb.wait()

    return hist_kernel(xt)


BB = 512
GH = BH // BB


def _mlp_body(_, counts_ref, emb_ref, w1_ref, b1_ref, w2t_ref, b2_ref, outt_ref):
    c = counts_ref[...]
    denom = 200.0 - c[:, 0:1]
    cv = c[:, :NCLS]
    col = lax.broadcasted_iota(jnp.int32, (BB, NCLS), 1)
    cv = jnp.where(col == 0, 0.0, cv)
    s = jnp.dot(
        cv.astype(jnp.bfloat16),
        emb_ref[...].astype(jnp.bfloat16),
        preferred_element_type=jnp.float32,
    )
    pooled = s / denom
    h = jnp.dot(
        pooled.astype(jnp.bfloat16),
        w1_ref[...].astype(jnp.bfloat16),
        preferred_element_type=jnp.float32,
    ) + b1_ref[...]
    h = jnp.maximum(h, 0.0)
    outt_ref[...] = lax.dot_general(
        w2t_ref[...].astype(jnp.bfloat16),
        h.astype(jnp.bfloat16),
        (((1,), (1,)), ((), ())),
        preferred_element_type=jnp.float32,
    ) + b2_ref[...]


def _tc_mlp_half(prev_outt, counts_h, emb, W1, b1r, W2t, b2r, half):
    body = _mlp_body if prev_outt is not None else (
        lambda c, e, w1, bb1, w2t, bb2, o: _mlp_body(None, c, e, w1, bb1, w2t, bb2, o)
    )
    specs = [
        pl.BlockSpec((BB, VPAD), lambda i: (i, 0)),
        pl.BlockSpec((NCLS, D), lambda i: (0, 0)),
        pl.BlockSpec((D, D), lambda i: (0, 0)),
        pl.BlockSpec((1, D), lambda i: (0, 0)),
        pl.BlockSpec((NCLS, D), lambda i: (0, 0)),
        pl.BlockSpec((NCLS, 1), lambda i: (0, 0)),
    ]
    args = (counts_h, emb, W1, b1r, W2t, b2r)
    aliases = {}
    if prev_outt is not None:
        specs = [pl.BlockSpec(memory_space=pl.MemorySpace.ANY)] + specs
        args = (prev_outt,) + args
        aliases = {0: 0}
    return pl.pallas_call(
        body,
        grid=(GH,),
        in_specs=specs,
        out_specs=pl.BlockSpec((NCLS, BB), lambda i, h=half: (0, h * GH + i)),
        out_shape=jax.ShapeDtypeStruct((NCLS, B), jnp.float32),
        input_output_aliases=aliases,
    )(*args)


def kernel(x, emb, W1, b1, W2, b2):
    xt = jnp.swapaxes(x.astype(jnp.int32), 0, 1)
    W2t = jnp.swapaxes(W2, 0, 1)
    b1r = b1.reshape(1, D)
    b2r = b2.reshape(NCLS, 1)

    counts = [_sc_histogram_half(xt, h) for h in range(NHALF)]
    outt = _tc_mlp_half(None, counts[0], emb, W1, b1r, W2t, b2r, 0)
    for h in range(1, NHALF):
        outt = _tc_mlp_half(outt, counts[h], emb, W1, b1r, W2t, b2r, h)
    return outt.T


# speedup vs baseline: 1.4086x; 1.0422x over previous
import functools

import jax
import jax.numpy as jnp
from jax import lax
from jax.experimental import pallas as pl
from jax.experimental.pallas import tpu as pltpu
from jax.experimental.pallas import tpu_sc as plsc

B = 4096
L = 200
D = 512
NCLS = 1000
VPAD = 1024

NW = 32
NHALF = 2
BH = B // NHALF
CH = BH // NW
KSH = 128 // CH


def _sc_histogram_half(xt, half):
    mesh = plsc.VectorSubcoreMesh(core_axis_name="c", subcore_axis_name="s")

    @functools.partial(
        pl.kernel,
        mesh=mesh,
        out_type=jax.ShapeDtypeStruct((BH, VPAD), jnp.float32),
        scratch_types=[
            pltpu.VMEM((L, 128), jnp.int32),
            pltpu.VMEM((CH, VPAD), jnp.float32),
            pltpu.SemaphoreType.DMA,
        ],
        compiler_params=pltpu.CompilerParams(needs_layout_passes=False),
    )
    def hist_kernel(xt_hbm, counts_hbm, idx_v, hist_v, sem):
        wid = lax.axis_index("c") * 16 + lax.axis_index("s")
        ones = jnp.ones((16,), jnp.float32)
        zeros = jnp.zeros((16,), jnp.float32)
        lane = lax.iota(jnp.int32, 16)

        aligned = half * BH + (wid // KSH) * 128
        off = (wid % KSH) * CH
        cp = pltpu.async_copy(xt_hbm.at[:, pl.ds(aligned, 128)], idx_v, sem)

        def zero_row(r, carry):
            for k in range(VPAD // 16):
                hist_v[r, pl.ds(k * 16, 16)] = zeros
            return carry

        lax.fori_loop(0, CH, zero_row, 0)
        cp.wait()

        wbs = []
        for g in range(CH // 16):
            rvec = lane + (g * 16)

            def do_tok(j, carry, rvec=rvec, g=g):
                for u in range(4):
                    ids = idx_v[j * 4 + u, pl.ds(off + g * 16, 16)]
                    plsc.addupdate_scatter(hist_v, [rvec, ids], ones)
                return carry

            lax.fori_loop(0, L // 4, do_tok, 0)
            wbs.append(pltpu.async_copy(
                hist_v.at[pl.ds(g * 16, 16)],
                counts_hbm.at[pl.ds(wid * CH + g * 16, 16)],
                sem,
            ))
        for wb in wbs:
            wb.wait()

    return hist_kernel(xt)


BB = 1024
GH = BH // BB


def _mlp_body(_, counts_ref, emb_ref, w1_ref, b1_ref, w2t_ref, b2_ref, outt_ref):
    c = counts_ref[...]
    denom = 200.0 - c[:, 0:1]
    cv = c[:, :NCLS]
    col = lax.broadcasted_iota(jnp.int32, (BB, NCLS), 1)
    cv = jnp.where(col == 0, 0.0, cv)
    s = jnp.dot(
        cv.astype(jnp.bfloat16),
        emb_ref[...].astype(jnp.bfloat16),
        preferred_element_type=jnp.float32,
    )
    pooled = s / denom
    h = jnp.dot(
        pooled.astype(jnp.bfloat16),
        w1_ref[...].astype(jnp.bfloat16),
        preferred_element_type=jnp.float32,
    ) + b1_ref[...]
    h = jnp.maximum(h, 0.0)
    outt_ref[...] = lax.dot_general(
        w2t_ref[...].astype(jnp.bfloat16),
        h.astype(jnp.bfloat16),
        (((1,), (1,)), ((), ())),
        preferred_element_type=jnp.float32,
    ) + b2_ref[...]


def _tc_mlp_half(prev_outt, counts_h, emb, W1, b1r, W2t, b2r, half):
    body = _mlp_body if prev_outt is not None else (
        lambda c, e, w1, bb1, w2t, bb2, o: _mlp_body(None, c, e, w1, bb1, w2t, bb2, o)
    )
    specs = [
        pl.BlockSpec((BB, VPAD), lambda i: (i, 0)),
        pl.BlockSpec((NCLS, D), lambda i: (0, 0)),
        pl.BlockSpec((D, D), lambda i: (0, 0)),
        pl.BlockSpec((1, D), lambda i: (0, 0)),
        pl.BlockSpec((NCLS, D), lambda i: (0, 0)),
        pl.BlockSpec((NCLS, 1), lambda i: (0, 0)),
    ]
    args = (counts_h, emb, W1, b1r, W2t, b2r)
    aliases = {}
    if prev_outt is not None:
        specs = [pl.BlockSpec(memory_space=pl.MemorySpace.ANY)] + specs
        args = (prev_outt,) + args
        aliases = {0: 0}
    return pl.pallas_call(
        body,
        grid=(GH,),
        in_specs=specs,
        out_specs=pl.BlockSpec((NCLS, BB), lambda i, h=half: (0, h * GH + i)),
        out_shape=jax.ShapeDtypeStruct((NCLS, B), jnp.float32),
        input_output_aliases=aliases,
    )(*args)


def kernel(x, emb, W1, b1, W2, b2):
    xt = jnp.swapaxes(x.astype(jnp.int32), 0, 1)
    W2t = jnp.swapaxes(W2, 0, 1)
    b1r = b1.reshape(1, D)
    b2r = b2.reshape(NCLS, 1)

    counts = [_sc_histogram_half(xt, h) for h in range(NHALF)]
    outt = _tc_mlp_half(None, counts[0], emb, W1, b1r, W2t, b2r, 0)
    for h in range(1, NHALF):
        outt = _tc_mlp_half(outt, counts[h], emb, W1, b1r, W2t, b2r, h)
    return outt.T
